# baseline (speedup 1.0000x reference)
import functools

import jax
import jax.numpy as jnp
from jax import lax
from jax.experimental import pallas as pl
from jax.experimental.pallas import tpu as pltpu
from jax.experimental.pallas import tpu_sc as plsc

VOCAB = 1_000_000
EMB = 128
OUT = 5
PAD_IDX = 0
B = 16384
L = 200

PW = 16
L_PAD = 208
IDX_MINOR = 104
NC, NS = 2, 16
NW = NC * NS
SEQ_W = B // NW
CH_SEQ = 8
NCH = SEQ_W // CH_SEQ
IDX_ROWS = CH_SEQ * (L_PAD // IDX_MINOR)
CH_ROWS = CH_SEQ * L_PAD
VB = 8000


PACK = EMB // PW


def _proj_body(t_ref, w2_ref, o_ref):
    q = jnp.dot(
        t_ref[...].astype(jnp.bfloat16),
        w2_ref[...],
        preferred_element_type=jnp.float32,
    )
    q3 = q.reshape(VB // PACK, PACK, EMB)
    lane = jax.lax.broadcasted_iota(jnp.int32, (VB // PACK, EMB), 1)
    prow = jax.lax.broadcasted_iota(jnp.int32, (VB // PACK, EMB), 0) + pl.program_id(0) * (VB // PACK)
    acc = jnp.zeros((VB // PACK, EMB), jnp.float32)
    for u in range(PACK):
        acc = acc + jnp.where(lane // PW == u, q3[:, u, :], 0.0)
    acc = jnp.where((prow == 0) & (lane < PW), 0.0, acc)
    o_ref[...] = acc


def _project(table, w2):
    return pl.pallas_call(
        _proj_body,
        grid=(VOCAB // VB,),
        in_specs=[
            pl.BlockSpec((VB, EMB), lambda i: (i, 0)),
            pl.BlockSpec((EMB, EMB), lambda i: (0, 0)),
        ],
        out_specs=pl.BlockSpec((VB // PACK, EMB), lambda i: (i, 0)),
        out_shape=jax.ShapeDtypeStruct((VOCAB // PACK, EMB), jnp.float32),
    )(table, w2)


@functools.partial(
    pl.kernel,
    out_type=jax.ShapeDtypeStruct((B, PW), jnp.float32),
    mesh=plsc.VectorSubcoreMesh(core_axis_name="c", subcore_axis_name="s"),
    compiler_params=pltpu.CompilerParams(use_tc_tiling_on_sc=False),
    scratch_types=[
        pltpu.VMEM((IDX_ROWS, IDX_MINOR), jnp.int32),
        pltpu.VMEM((IDX_ROWS, IDX_MINOR), jnp.int32),
        pltpu.VMEM((CH_ROWS, PW), jnp.float32),
        pltpu.VMEM((CH_ROWS, PW), jnp.float32),
        pltpu.VMEM((SEQ_W, PW), jnp.float32),
        pltpu.SemaphoreType.DMA,
    ],
)
def _sc_pool(idx_hbm, p_hbm, out_hbm, idx0, idx1, rows0, rows1, acc_v, sem):
    wid = lax.axis_index("s") * NC + lax.axis_index("c")
    idx_base = wid * (NCH * IDX_ROWS)

    def issue(c, idxbuf, rowbuf):
        pltpu.sync_copy(idx_hbm.at[pl.ds(idx_base + c * IDX_ROWS, IDX_ROWS)], idxbuf)
        for r in range(IDX_ROWS):
            pltpu.async_copy(
                p_hbm.at[idxbuf.at[r]],
                rowbuf.at[pl.ds(r * IDX_MINOR, IDX_MINOR)],
                sem,
            )

    def drain(rowbuf):
        pltpu.make_async_copy(p_hbm.at[pl.ds(0, CH_ROWS)], rowbuf, sem).wait()

    def seq_sum(rowbuf, s):
        base = s * L_PAD
        z = jnp.zeros((PW,), jnp.float32)

        def body(j, carry):
            a0, a1, a2, a3 = carry
            o = base + 4 * j
            return (
                a0 + rowbuf[o],
                a1 + rowbuf[o + 1],
                a2 + rowbuf[o + 2],
                a3 + rowbuf[o + 3],
            )

        a0, a1, a2, a3 = lax.fori_loop(0, L // 4, body, (z, z, z, z))
        return (a0 + a1) + (a2 + a3)

    bufs = ((idx0, rows0), (idx1, rows1))
    issue(0, idx0, rows0)

    def step(i, carry):
        for bsel in range(2):
            c = 2 * i + bsel
            rowbuf = bufs[bsel][1]
            nidx, nrows = bufs[bsel ^ 1]
            drain(rowbuf)

            @pl.when(c + 1 < NCH)
            def _():
                issue(c + 1, nidx, nrows)

            for s in range(CH_SEQ):
                acc_v[c * CH_SEQ + s] = seq_sum(rowbuf, s)
        return carry

    lax.fori_loop(0, NCH // 2, step, 0)
    pltpu.sync_copy(acc_v, out_hbm.at[pl.ds(wid * SEQ_W, SEQ_W)])


def _head_body(s_ref, b_ref, o_ref):
    logits = s_ref[...] + b_ref[...]
    m = jnp.max(logits, axis=-1, keepdims=True)
    z = logits - m
    o_ref[...] = z - jnp.log(jnp.sum(jnp.exp(z), axis=-1, keepdims=True))


def kernel(input, table, W, b):
    w_pad = jnp.zeros((EMB, PW), jnp.float32).at[:, :OUT].set(W * (1.0 / L))
    w2 = jnp.tile(w_pad, (1, PACK)).astype(jnp.bfloat16)
    p = jnp.reshape(_project(table, w2), (VOCAB, PW))

    idx_pad = jnp.pad(input.astype(jnp.int32), ((0, 0), (0, L_PAD - L)), mode="edge")
    idx_flat = idx_pad.reshape(NW * NCH * IDX_ROWS, IDX_MINOR)
    s = _sc_pool(idx_flat, p)

    b_pad = jnp.full((1, PW), -1e30, jnp.float32).at[0, :OUT].set(b)
    out = pl.pallas_call(
        _head_body,
        out_shape=jax.ShapeDtypeStruct((B, PW), jnp.float32),
    )(s, b_pad)
    return out[:, :OUT]


# speedup vs baseline: 1.1130x; 1.1130x over previous
import functools

import jax
import jax.numpy as jnp
from jax import lax
from jax.experimental import pallas as pl
from jax.experimental.pallas import tpu as pltpu
from jax.experimental.pallas import tpu_sc as plsc

VOCAB = 1_000_000
EMB = 128
OUT = 5
PAD_IDX = 0
B = 16384
L = 200

PW = 16
IDX_MINOR = 100
NC, NS = 2, 16
NW = NC * NS
SEQ_W = B // NW
CH_SEQ = 16
NCH = SEQ_W // CH_SEQ
IDX_ROWS = CH_SEQ * (L // IDX_MINOR)
CH_ROWS = CH_SEQ * L
VB = 40000


PACK = EMB // PW


def _proj_body(t_ref, w2_ref, o_ref):
    q = jnp.dot(t_ref[...], w2_ref[...], preferred_element_type=jnp.float32)
    q3 = q.reshape(VB // PACK, PACK, EMB)
    lane = jax.lax.broadcasted_iota(jnp.int32, (VB // PACK, EMB), 1)
    prow = jax.lax.broadcasted_iota(jnp.int32, (VB // PACK, EMB), 0) + pl.program_id(0) * (VB // PACK)
    acc = jnp.zeros((VB // PACK, EMB), jnp.float32)
    for u in range(PACK):
        acc = acc + jnp.where(lane // PW == u, q3[:, u, :], 0.0)
    acc = jnp.where((prow == 0) & (lane < PW), 0.0, acc)
    o_ref[...] = acc


def _project(table, w2):
    return pl.pallas_call(
        _proj_body,
        grid=(VOCAB // VB,),
        in_specs=[
            pl.BlockSpec((VB, EMB), lambda i: (i, 0)),
            pl.BlockSpec((EMB, EMB), lambda i: (0, 0)),
        ],
        out_specs=pl.BlockSpec((VB // PACK, EMB), lambda i: (i, 0)),
        out_shape=jax.ShapeDtypeStruct((VOCAB // PACK, EMB), jnp.float32),
    )(table, w2)


@functools.partial(
    pl.kernel,
    out_type=jax.ShapeDtypeStruct((B, PW), jnp.float32),
    mesh=plsc.VectorSubcoreMesh(core_axis_name="c", subcore_axis_name="s"),
    compiler_params=pltpu.CompilerParams(use_tc_tiling_on_sc=False),
    scratch_types=[
        pltpu.VMEM((IDX_ROWS, IDX_MINOR), jnp.int32),
        pltpu.VMEM((IDX_ROWS, IDX_MINOR), jnp.int32),
        pltpu.VMEM((CH_ROWS, PW), jnp.float32),
        pltpu.VMEM((CH_ROWS, PW), jnp.float32),
        pltpu.VMEM((SEQ_W, PW), jnp.float32),
        pltpu.SemaphoreType.DMA,
        pltpu.SemaphoreType.DMA,
    ],
)
def _sc_pool(idx_hbm, p_hbm, out_hbm, idx0, idx1, rows0, rows1, acc_v, sem_g, sem_i):
    wid = lax.axis_index("s") * NC + lax.axis_index("c")
    idx_base = wid * (NCH * IDX_ROWS)

    def idx_start(c, idxbuf):
        pltpu.async_copy(
            idx_hbm.at[pl.ds(idx_base + c * IDX_ROWS, IDX_ROWS)], idxbuf, sem_i
        )

    def idx_wait(idxbuf):
        pltpu.make_async_copy(idx_hbm.at[pl.ds(0, IDX_ROWS)], idxbuf, sem_i).wait()

    def gathers(idxbuf, rowbuf):
        for r in range(IDX_ROWS):
            pltpu.async_copy(
                p_hbm.at[idxbuf.at[r]],
                rowbuf.at[pl.ds(r * IDX_MINOR, IDX_MINOR)],
                sem_g,
            )

    def drain(rowbuf):
        pltpu.make_async_copy(p_hbm.at[pl.ds(0, CH_ROWS)], rowbuf, sem_g).wait()

    def seq_sum(rowbuf, s):
        base = s * L
        z = jnp.zeros((PW,), jnp.float32)

        def body(j, carry):
            a0, a1, a2, a3 = carry
            o = base + 4 * j
            return (
                a0 + rowbuf[o],
                a1 + rowbuf[o + 1],
                a2 + rowbuf[o + 2],
                a3 + rowbuf[o + 3],
            )

        a0, a1, a2, a3 = lax.fori_loop(0, L // 4, body, (z, z, z, z))
        return (a0 + a1) + (a2 + a3)

    bufs = ((idx0, rows0), (idx1, rows1))
    pltpu.sync_copy(idx_hbm.at[pl.ds(idx_base, IDX_ROWS)], idx0)
    gathers(idx0, rows0)
    idx_start(1, idx1)

    def step(i, carry):
        for bsel in range(2):
            c = 2 * i + bsel
            idxbuf, rowbuf = bufs[bsel]
            nidx, nrows = bufs[bsel ^ 1]
            drain(rowbuf)

            @pl.when(c + 1 < NCH)
            def _():
                idx_wait(nidx)
                gathers(nidx, nrows)

                @pl.when(c + 2 < NCH)
                def _():
                    idx_start(c + 2, idxbuf)

            for s in range(CH_SEQ):
                acc_v[c * CH_SEQ + s] = seq_sum(rowbuf, s)
        return carry

    lax.fori_loop(0, NCH // 2, step, 0)
    pltpu.sync_copy(acc_v, out_hbm.at[pl.ds(wid * SEQ_W, SEQ_W)])


def _head_body(s_ref, b_ref, o_ref):
    logits = s_ref[...] + b_ref[...]
    m = jnp.max(logits, axis=-1, keepdims=True)
    z = logits - m
    o_ref[...] = z - jnp.log(jnp.sum(jnp.exp(z), axis=-1, keepdims=True))


def kernel(input, table, W, b):
    w_pad = jnp.zeros((EMB, PW), jnp.float32).at[:, :OUT].set(W * (1.0 / L))
    w2 = jnp.tile(w_pad, (1, PACK))
    p = jnp.reshape(_project(table, w2), (VOCAB, PW))

    idx_flat = jnp.reshape(input.astype(jnp.int32), (NW * NCH * IDX_ROWS, IDX_MINOR))
    s = _sc_pool(idx_flat, p)

    b_pad = jnp.full((1, PW), -1e30, jnp.float32).at[0, :OUT].set(b)
    out = pl.pallas_call(
        _head_body,
        out_shape=jax.ShapeDtypeStruct((B, PW), jnp.float32),
    )(s, b_pad)
    return out[:, :OUT]


# speedup vs baseline: 1.1777x; 1.0582x over previous
import functools

import jax
import jax.numpy as jnp
from jax import lax
from jax.experimental import pallas as pl
from jax.experimental.pallas import tpu as pltpu
from jax.experimental.pallas import tpu_sc as plsc

VOCAB = 1_000_000
EMB = 128
OUT = 5
PAD_IDX = 0
B = 16384
L = 200

PW = 16
IDX_MINOR = 100
NC, NS = 2, 16
NW = NC * NS
SEQ_W = B // NW
CH_SEQ = 16
NCH = SEQ_W // CH_SEQ
IDX_ROWS = CH_SEQ * (L // IDX_MINOR)
CH_ROWS = CH_SEQ * L
VB = 40000


PACK = EMB // PW


def _proj_body(t_ref, w2_ref, o_ref):
    q = jnp.dot(t_ref[...], w2_ref[...], preferred_element_type=jnp.float32)
    q3 = q.reshape(VB // PACK, PACK, EMB)
    lane = jax.lax.broadcasted_iota(jnp.int32, (VB // PACK, EMB), 1)
    prow = jax.lax.broadcasted_iota(jnp.int32, (VB // PACK, EMB), 0) + pl.program_id(0) * (VB // PACK)
    acc = jnp.zeros((VB // PACK, EMB), jnp.float32)
    for u in range(PACK):
        acc = acc + jnp.where(lane // PW == u, q3[:, u, :], 0.0)
    acc = jnp.where((prow == 0) & (lane < PW), 0.0, acc)
    o_ref[...] = acc


def _project(table, w2):
    return pl.pallas_call(
        _proj_body,
        grid=(VOCAB // VB,),
        in_specs=[
            pl.BlockSpec((VB, EMB), lambda i: (i, 0)),
            pl.BlockSpec((EMB, EMB), lambda i: (0, 0)),
        ],
        out_specs=pl.BlockSpec((VB // PACK, EMB), lambda i: (i, 0)),
        out_shape=jax.ShapeDtypeStruct((VOCAB // PACK, EMB), jnp.float32),
    )(table, w2)


@functools.partial(
    pl.kernel,
    out_type=jax.ShapeDtypeStruct((B, PW), jnp.float32),
    mesh=plsc.VectorSubcoreMesh(core_axis_name="c", subcore_axis_name="s"),
    compiler_params=pltpu.CompilerParams(use_tc_tiling_on_sc=False),
    scratch_types=[
        pltpu.VMEM((CH_SEQ, L), jnp.int32),
        pltpu.VMEM((CH_SEQ, L), jnp.int32),
        pltpu.VMEM((CH_ROWS, PW), jnp.float32),
        pltpu.VMEM((CH_ROWS, PW), jnp.float32),
        pltpu.VMEM((SEQ_W, PW), jnp.float32),
        pltpu.SemaphoreType.DMA,
        pltpu.SemaphoreType.DMA,
    ],
)
def _sc_pool(idx_hbm, p_hbm, out_hbm, idx0, idx1, rows0, rows1, acc_v, sem_g, sem_i):
    wid = lax.axis_index("s") * NC + lax.axis_index("c")
    seq_base = wid * SEQ_W

    def idx_start(c, idxbuf):
        pltpu.async_copy(
            idx_hbm.at[pl.ds(seq_base + c * CH_SEQ, CH_SEQ)], idxbuf, sem_i
        )

    def idx_wait(idxbuf):
        pltpu.make_async_copy(idx_hbm.at[pl.ds(0, CH_SEQ)], idxbuf, sem_i).wait()

    def gathers(idxbuf, rowbuf):
        for r in range(CH_SEQ):
            for off, n in ((0, 96), (96, 104)):
                pltpu.async_copy(
                    p_hbm.at[idxbuf.at[r, pl.ds(off, n)]],
                    rowbuf.at[pl.ds(r * L + off, n)],
                    sem_g,
                )

    def drain(rowbuf):
        pltpu.make_async_copy(p_hbm.at[pl.ds(0, CH_ROWS)], rowbuf, sem_g).wait()

    def seq_sum(rowbuf, s):
        base = s * L
        z = jnp.zeros((PW,), jnp.float32)

        def body(j, carry):
            a0, a1, a2, a3 = carry
            o = base + 4 * j
            return (
                a0 + rowbuf[o],
                a1 + rowbuf[o + 1],
                a2 + rowbuf[o + 2],
                a3 + rowbuf[o + 3],
            )

        a0, a1, a2, a3 = lax.fori_loop(0, L // 4, body, (z, z, z, z))
        return (a0 + a1) + (a2 + a3)

    bufs = ((idx0, rows0), (idx1, rows1))
    pltpu.sync_copy(idx_hbm.at[pl.ds(seq_base, CH_SEQ)], idx0)
    gathers(idx0, rows0)
    idx_start(1, idx1)

    def step(i, carry):
        for bsel in range(2):
            c = 2 * i + bsel
            idxbuf, rowbuf = bufs[bsel]
            nidx, nrows = bufs[bsel ^ 1]
            drain(rowbuf)

            @pl.when(c + 1 < NCH)
            def _():
                idx_wait(nidx)
                gathers(nidx, nrows)

                @pl.when(c + 2 < NCH)
                def _():
                    idx_start(c + 2, idxbuf)

            for s in range(CH_SEQ):
                acc_v[c * CH_SEQ + s] = seq_sum(rowbuf, s)
        return carry

    lax.fori_loop(0, NCH // 2, step, 0)
    pltpu.sync_copy(acc_v, out_hbm.at[pl.ds(wid * SEQ_W, SEQ_W)])


def _head_body(s_ref, b_ref, o_ref):
    logits = s_ref[...] + b_ref[...]
    m = jnp.max(logits, axis=-1, keepdims=True)
    z = logits - m
    o_ref[...] = z - jnp.log(jnp.sum(jnp.exp(z), axis=-1, keepdims=True))


def kernel(input, table, W, b):
    w_pad = jnp.zeros((EMB, PW), jnp.float32).at[:, :OUT].set(W * (1.0 / L))
    w2 = jnp.tile(w_pad, (1, PACK))
    p = jnp.reshape(_project(table, w2), (VOCAB, PW))

    s = _sc_pool(input, p)

    b_pad = jnp.full((1, PW), -1e30, jnp.float32).at[0, :OUT].set(b)
    out = pl.pallas_call(
        _head_body,
        out_shape=jax.ShapeDtypeStruct((B, PW), jnp.float32),
    )(s, b_pad)
    return out[:, :OUT]


# speedup vs baseline: 1.2140x; 1.0308x over previous
import functools

import jax
import jax.numpy as jnp
from jax import lax
from jax.experimental import pallas as pl
from jax.experimental.pallas import tpu as pltpu
from jax.experimental.pallas import tpu_sc as plsc

VOCAB = 1_000_000
EMB = 128
OUT = 5
PAD_IDX = 0
B = 16384
L = 200

PW = 16
IDX_MINOR = 100
NC, NS = 2, 16
NW = NC * NS
SEQ_W = B // NW
CH_SEQ = 16
NCH = SEQ_W // CH_SEQ
IDX_ROWS = CH_SEQ * (L // IDX_MINOR)
CH_ROWS = CH_SEQ * L
VB = 40000


PACK = EMB // PW


def _proj_body(t_ref, w2_ref, o_ref):
    q = jnp.dot(t_ref[...], w2_ref[...], preferred_element_type=jnp.float32)
    q3 = q.reshape(VB // PACK, PACK, EMB)
    lane = jax.lax.broadcasted_iota(jnp.int32, (VB // PACK, EMB), 1)
    prow = jax.lax.broadcasted_iota(jnp.int32, (VB // PACK, EMB), 0) + pl.program_id(0) * (VB // PACK)
    acc = jnp.zeros((VB // PACK, EMB), jnp.float32)
    for u in range(PACK):
        acc = acc + jnp.where(lane // PW == u, q3[:, u, :], 0.0)
    acc = jnp.where((prow == 0) & (lane < PW), 0.0, acc)
    o_ref[...] = acc


def _project(table, w2):
    return pl.pallas_call(
        _proj_body,
        grid=(VOCAB // VB,),
        in_specs=[
            pl.BlockSpec((VB, EMB), lambda i: (i, 0)),
            pl.BlockSpec((EMB, EMB), lambda i: (0, 0)),
        ],
        out_specs=pl.BlockSpec((VB // PACK, EMB), lambda i: (i, 0)),
        out_shape=jax.ShapeDtypeStruct((VOCAB // PACK, EMB), jnp.float32),
    )(table, w2)


@functools.partial(
    pl.kernel,
    out_type=jax.ShapeDtypeStruct((B, PW), jnp.float32),
    mesh=plsc.VectorSubcoreMesh(core_axis_name="c", subcore_axis_name="s"),
    compiler_params=pltpu.CompilerParams(use_tc_tiling_on_sc=False),
    scratch_types=[
        pltpu.VMEM((CH_SEQ, L), jnp.int32),
        pltpu.VMEM((CH_SEQ, L), jnp.int32),
        pltpu.VMEM((CH_ROWS, PW), jnp.float32),
        pltpu.VMEM((CH_ROWS, PW), jnp.float32),
        pltpu.VMEM((SEQ_W, PW), jnp.float32),
        pltpu.SemaphoreType.DMA,
        pltpu.SemaphoreType.DMA,
        pltpu.SemaphoreType.DMA,
    ],
)
def _sc_pool(idx_hbm, p_hbm, out_hbm, idx0, idx1, rows0, rows1, acc_v, sem_g0, sem_g1, sem_i):
    wid = lax.axis_index("s") * NC + lax.axis_index("c")
    seq_base = wid * SEQ_W

    def idx_start(c, idxbuf):
        pltpu.async_copy(
            idx_hbm.at[pl.ds(seq_base + c * CH_SEQ, CH_SEQ)], idxbuf, sem_i
        )

    def idx_wait(idxbuf):
        pltpu.make_async_copy(idx_hbm.at[pl.ds(0, CH_SEQ)], idxbuf, sem_i).wait()

    def gathers(idxbuf, rowbuf, sem):
        for r in range(CH_SEQ):
            for off, n in ((0, 96), (96, 104)):
                pltpu.async_copy(
                    p_hbm.at[idxbuf.at[r, pl.ds(off, n)]],
                    rowbuf.at[pl.ds(r * L + off, n)],
                    sem,
                )

    def drain(rowbuf, sem):
        pltpu.make_async_copy(p_hbm.at[pl.ds(0, CH_ROWS)], rowbuf, sem).wait()

    def seq_sum(rowbuf, s):
        base = s * L
        z = jnp.zeros((PW,), jnp.float32)

        def body(j, carry):
            a0, a1, a2, a3 = carry
            o = base + 4 * j
            return (
                a0 + rowbuf[o],
                a1 + rowbuf[o + 1],
                a2 + rowbuf[o + 2],
                a3 + rowbuf[o + 3],
            )

        a0, a1, a2, a3 = lax.fori_loop(0, L // 4, body, (z, z, z, z))
        return (a0 + a1) + (a2 + a3)

    bufs = ((idx0, rows0, sem_g0), (idx1, rows1, sem_g1))
    pltpu.sync_copy(idx_hbm.at[pl.ds(seq_base, CH_SEQ)], idx0)
    gathers(idx0, rows0, sem_g0)
    idx_start(1, idx1)

    def step(i, carry):
        for bsel in range(2):
            c = 2 * i + bsel
            idxbuf, rowbuf, sg = bufs[bsel]
            nidx, nrows, nsg = bufs[bsel ^ 1]

            @pl.when(c + 1 < NCH)
            def _():
                idx_wait(nidx)
                gathers(nidx, nrows, nsg)

            drain(rowbuf, sg)

            @pl.when(c + 2 < NCH)
            def _():
                idx_start(c + 2, idxbuf)

            for s in range(CH_SEQ):
                acc_v[c * CH_SEQ + s] = seq_sum(rowbuf, s)
        return carry

    lax.fori_loop(0, NCH // 2, step, 0)
    pltpu.sync_copy(acc_v, out_hbm.at[pl.ds(wid * SEQ_W, SEQ_W)])


def _head_body(s_ref, b_ref, o_ref):
    logits = s_ref[...] + b_ref[...]
    m = jnp.max(logits, axis=-1, keepdims=True)
    z = logits - m
    o_ref[...] = (z - jnp.log(jnp.sum(jnp.exp(z), axis=-1, keepdims=True)))[:, :OUT]


def kernel(input, table, W, b):
    w_pad = jnp.zeros((EMB, PW), jnp.float32).at[:, :OUT].set(W * (1.0 / L))
    w2 = jnp.tile(w_pad, (1, PACK))
    p = jnp.reshape(_project(table, w2), (VOCAB, PW))

    s = _sc_pool(input, p)

    b_pad = jnp.full((1, PW), -1e30, jnp.float32).at[0, :OUT].set(b)
    return pl.pallas_call(
        _head_body,
        out_shape=jax.ShapeDtypeStruct((B, OUT), jnp.float32),
    )(s, b_pad)
